# Initial kernel scaffold; baseline (speedup 1.0000x reference)
import jax
import jax.numpy as jnp
from jax.experimental import pallas as pl


def kernel(x, table, pos_encoding):
    raise NotImplementedError("write your pallas kernel here")


# speedup vs baseline: 1.0583x; 1.0583x over previous
import functools

import jax
import jax.numpy as jnp
from jax import lax
from jax.experimental import pallas as pl
from jax.experimental.pallas import tpu as pltpu
from jax.experimental.pallas import tpu_sc as plsc

_CHUNK = 128


def kernel(x, table, pos_encoding):
    B, T = x.shape
    V, D = table.shape
    N = B * T

    info = plsc.get_sparse_core_info()
    NC, NS, L = info.num_cores, info.num_subcores, info.num_lanes
    NW = NC * NS
    bpw = N // NW
    n_chunks = bpw // _CHUNK

    mesh = plsc.VectorSubcoreMesh(core_axis_name="c", subcore_axis_name="s")

    @functools.partial(
        pl.kernel,
        mesh=mesh,
        out_type=jax.ShapeDtypeStruct((N, D), jnp.float32),
        scratch_types=[
            pltpu.VMEM((n_chunks, _CHUNK), jnp.int32),
            pltpu.VMEM((bpw, D), jnp.float32),
            pltpu.VMEM((bpw, D), jnp.float32),
            pltpu.SemaphoreType.DMA,
        ],
    )
    def emb_body(x_hbm, table_hbm, pos_hbm, out_hbm, idx_v, rows_v, pos_v, sem):
        wid = lax.axis_index("s") * NC + lax.axis_index("c")
        base = wid * bpw
        pos_base = lax.rem(base, T)

        pltpu.sync_copy(x_hbm.at[pl.ds(wid * n_chunks, n_chunks)], idx_v)
        gathers = [
            pltpu.async_copy(
                table_hbm.at[idx_v.at[j]],
                rows_v.at[pl.ds(j * _CHUNK, _CHUNK)],
                sem,
            )
            for j in range(n_chunks)
        ]
        pltpu.sync_copy(pos_hbm.at[pl.ds(pos_base, bpw)], pos_v)
        for g in gathers:
            g.wait()

        def row_add(r, carry):
            for c in range(D // L):
                sl = pl.ds(c * L, L)
                rows_v[r, sl] = rows_v[r, sl] + pos_v[r, sl]
            return carry

        lax.fori_loop(0, bpw, row_add, 0)
        pltpu.sync_copy(rows_v, out_hbm.at[pl.ds(base, bpw)])

    out = emb_body(x.reshape(N // _CHUNK, _CHUNK), table, pos_encoding)
    return out.reshape(B, T, D)


# speedup vs baseline: 1.1083x; 1.0472x over previous
import functools

import jax
import jax.numpy as jnp
from jax import lax
from jax.experimental import pallas as pl
from jax.experimental.pallas import tpu as pltpu
from jax.experimental.pallas import tpu_sc as plsc


def kernel(x, table, pos_encoding):
    B, T = x.shape
    V, D = table.shape
    N = B * T

    info = plsc.get_sparse_core_info()
    NC, NS, L = info.num_cores, info.num_subcores, info.num_lanes
    NW = NC * NS
    PW = T // NW

    mesh = plsc.VectorSubcoreMesh(core_axis_name="c", subcore_axis_name="s")

    @functools.partial(
        pl.kernel,
        mesh=mesh,
        out_type=jax.ShapeDtypeStruct((N, D), jnp.float32),
        scratch_types=[
            pltpu.VMEM((B * PW,), jnp.int32),
            pltpu.VMEM((B * PW, D), jnp.float32),
            pltpu.VMEM((PW, D), jnp.float32),
            pltpu.SemaphoreType.DMA,
            [pltpu.SemaphoreType.DMA] * B,
            pltpu.SemaphoreType.DMA,
            pltpu.SemaphoreType.DMA,
        ],
    )
    def emb_body(x_hbm, table_hbm, pos_hbm, out_hbm,
                 idx_v, rows_v, pos_v, pos_sem, row_sems, out_sem, idx_sem):
        wid = lax.axis_index("s") * NC + lax.axis_index("c")
        pbase = wid * PW

        idx_cps = [
            pltpu.async_copy(
                x_hbm.at[pl.ds(b * T + pbase, PW)],
                idx_v.at[pl.ds(b * PW, PW)],
                idx_sem,
            )
            for b in range(B)
        ]
        for cp in idx_cps:
            cp.wait()
        gathers = [
            pltpu.async_copy(
                table_hbm.at[idx_v.at[pl.ds(b * PW, PW)]],
                rows_v.at[pl.ds(b * PW, PW)],
                row_sems[b],
            )
            for b in range(B)
        ]
        pos_cp = pltpu.async_copy(pos_hbm.at[pl.ds(pbase, PW)], pos_v, pos_sem)

        pos_cp.wait()
        outs = []
        for b in range(B):
            gathers[b].wait()
            rbase = b * PW

            @plsc.parallel_loop(0, PW, 1, unroll=2)
            def add_row(r):
                for c in range(D // L):
                    sl = pl.ds(c * L, L)
                    rows_v[rbase + r, sl] = rows_v[rbase + r, sl] + pos_v[r, sl]

            outs.append(pltpu.async_copy(
                rows_v.at[pl.ds(rbase, PW)],
                out_hbm.at[pl.ds(b * T + pbase, PW)],
                out_sem,
            ))
        for o in outs:
            o.wait()

    out = emb_body(x.reshape(N), table, pos_encoding)
    return out.reshape(B, T, D)
